# Initial kernel scaffold; baseline (speedup 1.0000x reference)
import jax
import jax.numpy as jnp
from jax.experimental import pallas as pl


def kernel(text, emb, W1, b1, W2, b2, W3, b3):
    raise NotImplementedError("write your pallas kernel here")


# speedup vs baseline: 10.7684x; 10.7684x over previous
import functools

import jax
import jax.numpy as jnp
from jax import lax
from jax.experimental import pallas as pl
from jax.experimental.pallas import tpu as pltpu
from jax.experimental.pallas import tpu_sc as plsc

VOCAB = 100000
EMB = 128
B = 4096
L = 200
H1 = 1024
H2 = 512
DOUT = 64

NC = 2
NS = 16
NW = NC * NS
LANE = 16
BPW = B // NW
CHUNK = 100
NCHUNK = BPW * L // CHUNK
NVEC = EMB // LANE


def _pool_body(idx_hbm, emb_hbm, out_hbm, idx_v, rows_v, acc_v, sem0, sem1):
    wid = lax.axis_index("s") * NC + lax.axis_index("c")
    pltpu.sync_copy(idx_hbm.at[wid], idx_v)

    sems = (sem0, sem1)

    def start(c, k):
        pltpu.async_copy(emb_hbm.at[idx_v.at[c]], rows_v.at[k], sems[k])

    def wait(c, k):
        pltpu.make_async_copy(
            emb_hbm.at[idx_v.at[c]], rows_v.at[k], sems[k]
        ).wait()

    def accum(k, acc):
        buf = rows_v.at[k]

        def row(l, acc):
            return tuple(
                acc[j] + buf[l, pl.ds(LANE * j, LANE)] for j in range(NVEC)
            )

        return lax.fori_loop(0, CHUNK, row, acc, unroll=2)

    start(0, 0)

    scale = jnp.float32(1.0 / L)

    def batch_row(b, carry):
        acc = tuple(jnp.zeros((LANE,), jnp.float32) for _ in range(NVEC))
        for k in (0, 1):
            c = 2 * b + k
            nxt = c + 1

            @pl.when(nxt < NCHUNK)
            def _():
                start(nxt, (k + 1) % 2)

            wait(c, k)
            acc = accum(k, acc)
        for j in range(NVEC):
            acc_v[b, pl.ds(LANE * j, LANE)] = acc[j] * scale
        return carry

    lax.fori_loop(0, BPW, batch_row, 0)
    pltpu.sync_copy(acc_v, out_hbm.at[pl.ds(wid * BPW, BPW)])


@functools.partial(
    pl.kernel,
    out_type=jax.ShapeDtypeStruct((B, EMB), jnp.float32),
    mesh=plsc.VectorSubcoreMesh(core_axis_name="c", subcore_axis_name="s"),
    scratch_types=[
        pltpu.VMEM((NCHUNK, CHUNK), jnp.int32),
        pltpu.VMEM((2, CHUNK, EMB), jnp.float32),
        pltpu.VMEM((BPW, EMB), jnp.float32),
        pltpu.SemaphoreType.DMA,
        pltpu.SemaphoreType.DMA,
    ],
)
def _pool(idx_hbm, emb_hbm, out_hbm, idx_v, rows_v, acc_v, sem0, sem1):
    _pool_body(idx_hbm, emb_hbm, out_hbm, idx_v, rows_v, acc_v, sem0, sem1)


MLP_BB = 512


def _mlp_body(x_ref, w1_ref, b1_ref, w2_ref, b2_ref, w3_ref, b3_ref, o_ref):
    dn = (((1,), (1,)), ((), ()))
    x = x_ref[...]
    h = lax.dot_general(x, w1_ref[...], dn, preferred_element_type=jnp.float32)
    h = jnp.maximum(h + b1_ref[...], 0.0)
    h = lax.dot_general(h, w2_ref[...], dn, preferred_element_type=jnp.float32)
    h = jnp.maximum(h + b2_ref[...], 0.0)
    h = lax.dot_general(h, w3_ref[...], dn, preferred_element_type=jnp.float32)
    o_ref[...] = h + b3_ref[...]


def _mlp(x, W1, b1, W2, b2, W3, b3):
    rep2 = lambda i: (0, 0)
    return pl.pallas_call(
        _mlp_body,
        grid=(B // MLP_BB,),
        in_specs=[
            pl.BlockSpec((MLP_BB, EMB), lambda i: (i, 0)),
            pl.BlockSpec((H1, EMB), rep2),
            pl.BlockSpec((1, H1), rep2),
            pl.BlockSpec((H2, H1), rep2),
            pl.BlockSpec((1, H2), rep2),
            pl.BlockSpec((DOUT, H2), rep2),
            pl.BlockSpec((1, DOUT), rep2),
        ],
        out_specs=pl.BlockSpec((MLP_BB, DOUT), lambda i: (i, 0)),
        out_shape=jax.ShapeDtypeStruct((B, DOUT), jnp.float32),
    )(x, W1, b1.reshape(1, H1), W2, b2.reshape(1, H2), W3, b3.reshape(1, DOUT))


def kernel(text, emb, W1, b1, W2, b2, W3, b3):
    idx = text.astype(jnp.int32).reshape(NW, NCHUNK, CHUNK)
    pooled = _pool(idx, emb)
    return _mlp(pooled, W1, b1, W2, b2, W3, b3)


# speedup vs baseline: 15.4309x; 1.4330x over previous
import functools

import jax
import jax.numpy as jnp
from jax import lax
from jax.experimental import pallas as pl
from jax.experimental.pallas import tpu as pltpu
from jax.experimental.pallas import tpu_sc as plsc

VOCAB = 100000
EMB = 128
B = 4096
L = 200
H1 = 1024
H2 = 512
DOUT = 64

NC = 2
NS = 16
NW = NC * NS
LANE = 16
BPW = B // NW
CHUNK = 100
NCHUNK = BPW * L // CHUNK
NVEC = EMB // LANE


NBUF = 4


def _pool_body(idx_hbm, emb_hbm, out_hbm, idx_v, rows_v, acc_v, *sems):
    wid = lax.axis_index("s") * NC + lax.axis_index("c")
    pltpu.sync_copy(idx_hbm.at[wid], idx_v)

    def start(c, k):
        pltpu.async_copy(emb_hbm.at[idx_v.at[c]], rows_v.at[k], sems[k])

    def wait(c, k):
        pltpu.make_async_copy(
            emb_hbm.at[idx_v.at[c]], rows_v.at[k], sems[k]
        ).wait()

    def accum(k, acc):
        buf = rows_v.at[k]

        def row(l, acc):
            return tuple(
                acc[j] + buf[l, pl.ds(LANE * j, LANE)] for j in range(NVEC)
            )

        return lax.fori_loop(0, CHUNK, row, acc, unroll=5)

    for k in range(NBUF - 1):
        start(k, k)

    scale = jnp.float32(1.0 / L)

    def batch_pair(p, carry):
        for b2 in range(NBUF // 2):
            b = (NBUF // 2) * p + b2
            acc = tuple(jnp.zeros((LANE,), jnp.float32) for _ in range(NVEC))
            for k in (0, 1):
                c = NBUF * p + 2 * b2 + k
                buf = 2 * b2 + k
                nxt = c + NBUF - 1

                @pl.when(nxt < NCHUNK)
                def _():
                    start(nxt, (buf + NBUF - 1) % NBUF)

                wait(c, buf)
                acc = accum(buf, acc)
            for j in range(NVEC):
                acc_v[b, pl.ds(LANE * j, LANE)] = acc[j] * scale
        return carry

    lax.fori_loop(0, BPW // (NBUF // 2), batch_pair, 0)
    pltpu.sync_copy(acc_v, out_hbm.at[pl.ds(wid * BPW, BPW)])


@functools.partial(
    pl.kernel,
    out_type=jax.ShapeDtypeStruct((B, EMB), jnp.float32),
    mesh=plsc.VectorSubcoreMesh(core_axis_name="c", subcore_axis_name="s"),
    scratch_types=[
        pltpu.VMEM((NCHUNK, CHUNK), jnp.int32),
        pltpu.VMEM((NBUF, CHUNK, EMB), jnp.float32),
        pltpu.VMEM((BPW, EMB), jnp.float32),
    ] + [pltpu.SemaphoreType.DMA] * NBUF,
)
def _pool(idx_hbm, emb_hbm, out_hbm, idx_v, rows_v, acc_v, *sems):
    _pool_body(idx_hbm, emb_hbm, out_hbm, idx_v, rows_v, acc_v, *sems)


MLP_BB = 512


def _mlp_body(x_ref, w1_ref, b1_ref, w2_ref, b2_ref, w3_ref, b3_ref, o_ref):
    dn = (((1,), (1,)), ((), ()))
    x = x_ref[...]
    h = lax.dot_general(x, w1_ref[...], dn, preferred_element_type=jnp.float32)
    h = jnp.maximum(h + b1_ref[...], 0.0)
    h = lax.dot_general(h, w2_ref[...], dn, preferred_element_type=jnp.float32)
    h = jnp.maximum(h + b2_ref[...], 0.0)
    h = lax.dot_general(h, w3_ref[...], dn, preferred_element_type=jnp.float32)
    o_ref[...] = h + b3_ref[...]


def _mlp(x, W1, b1, W2, b2, W3, b3):
    rep2 = lambda i: (0, 0)
    return pl.pallas_call(
        _mlp_body,
        grid=(B // MLP_BB,),
        in_specs=[
            pl.BlockSpec((MLP_BB, EMB), lambda i: (i, 0)),
            pl.BlockSpec((H1, EMB), rep2),
            pl.BlockSpec((1, H1), rep2),
            pl.BlockSpec((H2, H1), rep2),
            pl.BlockSpec((1, H2), rep2),
            pl.BlockSpec((DOUT, H2), rep2),
            pl.BlockSpec((1, DOUT), rep2),
        ],
        out_specs=pl.BlockSpec((MLP_BB, DOUT), lambda i: (i, 0)),
        out_shape=jax.ShapeDtypeStruct((B, DOUT), jnp.float32),
    )(x, W1, b1.reshape(1, H1), W2, b2.reshape(1, H2), W3, b3.reshape(1, DOUT))


def kernel(text, emb, W1, b1, W2, b2, W3, b3):
    idx = text.astype(jnp.int32).reshape(NW, NCHUNK, CHUNK)
    pooled = _pool(idx, emb)
    return _mlp(pooled, W1, b1, W2, b2, W3, b3)
